# baseline (speedup 1.0000x reference)
import functools

import jax
import jax.numpy as jnp
from jax import lax
from jax.experimental import pallas as pl
from jax.experimental.pallas import tpu as pltpu
from jax.experimental.pallas import tpu_sc as plsc

_K = 40
_B = 4
_C = 3
_N = 4096
_ROWS = 256

_NC = 2
_NS = 16
_NW = _NC * _NS
_RPW = _N // _NW
_L = 16


_CH = 512
_NCHUNK = _N // _CH


def _knn_body(xb_ref, xall_ref, idx_ref):
    xb = xb_ref[0]
    xa = xall_ref[0]
    neg2inner = -2.0 * lax.dot_general(
        xb, xa, (((0,), (0,)), ((), ())), preferred_element_type=jnp.float32
    )
    xx_r = jnp.sum(xb * xb, axis=0)
    xx_c = jnp.sum(xa * xa, axis=0)
    d = (-xx_r[:, None] - neg2inner) - xx_c[None, :]
    d3 = d.reshape(_ROWS, _NCHUNK, _CH)
    lane = lax.broadcasted_iota(jnp.int32, (_ROWS, _CH), 1)
    iota_c = lax.broadcasted_iota(jnp.int32, (_ROWS, _NCHUNK), 1)
    g3 = (
        lax.broadcasted_iota(jnp.int32, (_ROWS, _NCHUNK, _CH), 1) * _CH
        + lax.broadcasted_iota(jnp.int32, (_ROWS, _NCHUNK, _CH), 2)
    )
    big = jnp.int32(1 << 30)
    neginf = jnp.float32(-jnp.inf)
    M = jnp.max(d3, axis=2)
    A = jnp.min(jnp.where(d3 == M[:, :, None], g3, big), axis=2)
    for k in range(_K):
        m = jnp.max(M, axis=1)
        amin = jnp.min(jnp.where(M == m[:, None], A, big), axis=1)
        idx_ref[0, :, k : k + 1] = amin[:, None]
        cstar = lax.shift_right_logical(amin, 9)
        ci = jnp.broadcast_to(cstar[:, None, None], (_ROWS, 1, _CH))
        g = jnp.take_along_axis(d3, ci, axis=1)[:, 0, :]
        gl = cstar[:, None] * _CH + lane
        keep = (g < m[:, None]) | ((g == m[:, None]) & (gl > amin[:, None]))
        gm = jnp.where(keep, g, neginf)
        nm = jnp.max(gm, axis=1)
        nA = jnp.min(
            jnp.where((gm == nm[:, None]) & keep, gl, big), axis=1
        )
        hit = iota_c == cstar[:, None]
        M = jnp.where(hit, nm[:, None], M)
        A = jnp.where(hit, nA[:, None], A)


def _topk_indices(x):
    return pl.pallas_call(
        _knn_body,
        grid=(_B, _N // _ROWS),
        in_specs=[
            pl.BlockSpec((1, _C, _ROWS), lambda b, r: (b, 0, r)),
            pl.BlockSpec((1, _C, _N), lambda b, r: (b, 0, 0)),
        ],
        out_specs=pl.BlockSpec((1, _ROWS, _K), lambda b, r: (b, r, 0)),
        out_shape=jax.ShapeDtypeStruct((_B, _N, _K), jnp.int32),
    )(x, x)


_PW = _RPW * _K


def _sc_gather_body(x_hbm, idx_hbm, out_hbm, table_v, idx_v, out_v):
    wid = lax.axis_index("s") * _NC + lax.axis_index("c")
    n0 = wid * _RPW
    p0 = wid * _PW
    lane = lax.iota(jnp.int32, _L)
    for b in range(_B):
        pltpu.sync_copy(x_hbm.at[pl.ds(b * _C * _N, _C * _N)], table_v)
        pltpu.sync_copy(idx_hbm.at[pl.ds(b * _N * _K + p0, _PW)], idx_v)

        def body(ci, carry):
            base = ci * _L
            pos = base + lane
            r = lax.div(pos, jnp.int32(_K))
            g = r + n0
            nidx = idx_v[pl.ds(base, _L)]
            for c in range(_C):
                off = jnp.int32(c * _N)
                nbr = plsc.load_gather(table_v, [off + nidx])
                ctr = plsc.load_gather(table_v, [off + g])
                out_v[pl.ds(c * _PW + base, _L)] = nbr - ctr
                out_v[pl.ds((c + _C) * _PW + base, _L)] = ctr
            return carry

        lax.fori_loop(0, _PW // _L, body, 0)
        for c in range(2 * _C):
            pltpu.sync_copy(
                out_v.at[pl.ds(c * _PW, _PW)],
                out_hbm.at[pl.ds((b * 2 * _C + c) * _N * _K + p0, _PW)],
            )


def _gather_features(x, idx):
    mesh = plsc.VectorSubcoreMesh(core_axis_name="c", subcore_axis_name="s")
    f = functools.partial(
        pl.kernel,
        mesh=mesh,
        compiler_params=pltpu.CompilerParams(needs_layout_passes=False),
        out_type=jax.ShapeDtypeStruct((_B * 2 * _C * _N * _K,), jnp.float32),
        scratch_types=[
            pltpu.VMEM((_C * _N,), jnp.float32),
            pltpu.VMEM((_PW,), jnp.int32),
            pltpu.VMEM((2 * _C * _PW,), jnp.float32),
        ],
    )(_sc_gather_body)
    out = f(x.reshape(-1), idx.reshape(-1))
    return out.reshape(_B, 2 * _C, _N, _K)


@jax.jit
def kernel(x, class_label):
    del class_label
    idx = _topk_indices(x)
    return _gather_features(x, idx)


# speedup vs baseline: 2.8493x; 2.8493x over previous
import functools

import jax
import jax.numpy as jnp
from jax import lax
from jax.experimental import pallas as pl
from jax.experimental.pallas import tpu as pltpu
from jax.experimental.pallas import tpu_sc as plsc

_K = 40
_B = 4
_C = 3
_N = 4096
_ROWS = 512

_NC = 2
_NS = 16
_NW = _NC * _NS
_RPW = _N // _NW
_L = 16


def _knn_body(xb_ref, xall_ref, idx_ref):
    xb = xb_ref[0]
    xa = xall_ref[0]
    neg2inner = -2.0 * lax.dot_general(
        xb, xa, (((0,), (0,)), ((), ())), preferred_element_type=jnp.float32
    )
    xx_r = jnp.sum(xb * xb, axis=0)
    xx_c = jnp.sum(xa * xa, axis=0)
    d = (-xx_r[:, None] - neg2inner) - xx_c[None, :]
    iota = lax.broadcasted_iota(jnp.int32, (_ROWS, _N), 1)
    big = jnp.int32(_N)
    neginf = jnp.float32(-jnp.inf)
    for k in range(_K):
        m = jnp.max(d, axis=1)
        cand = jnp.where(d == m[:, None], iota, big)
        amin = jnp.min(cand, axis=1)
        idx_ref[0, :, k : k + 1] = amin[:, None]
        d = jnp.where(cand == amin[:, None], neginf, d)


def _topk_indices(x):
    return pl.pallas_call(
        _knn_body,
        grid=(_B, _N // _ROWS),
        in_specs=[
            pl.BlockSpec((1, _C, _ROWS), lambda b, r: (b, 0, r)),
            pl.BlockSpec((1, _C, _N), lambda b, r: (b, 0, 0)),
        ],
        out_specs=pl.BlockSpec((1, _ROWS, _K), lambda b, r: (b, r, 0)),
        out_shape=jax.ShapeDtypeStruct((_B, _N, _K), jnp.int32),
    )(x, x)


_PW = _RPW * _K


def _sc_gather_body(x_hbm, idx_hbm, out_hbm, table_v, idx_v, out_v):
    wid = lax.axis_index("s") * _NC + lax.axis_index("c")
    n0 = wid * _RPW
    p0 = wid * _PW
    lane = lax.iota(jnp.int32, _L)
    for b in range(_B):
        pltpu.sync_copy(x_hbm.at[pl.ds(b * _C * _N, _C * _N)], table_v)
        pltpu.sync_copy(idx_hbm.at[pl.ds(b * _N * _K + p0, _PW)], idx_v)

        def body(ci, carry):
            base = ci * _L
            pos = base + lane
            r = lax.div(pos, jnp.int32(_K))
            g = r + n0
            nidx = idx_v[pl.ds(base, _L)]
            for c in range(_C):
                off = jnp.int32(c * _N)
                nbr = plsc.load_gather(table_v, [off + nidx])
                ctr = plsc.load_gather(table_v, [off + g])
                out_v[pl.ds(c * _PW + base, _L)] = nbr - ctr
                out_v[pl.ds((c + _C) * _PW + base, _L)] = ctr
            return carry

        lax.fori_loop(0, _PW // _L, body, 0)
        for c in range(2 * _C):
            pltpu.sync_copy(
                out_v.at[pl.ds(c * _PW, _PW)],
                out_hbm.at[pl.ds((b * 2 * _C + c) * _N * _K + p0, _PW)],
            )


def _gather_features(x, idx):
    mesh = plsc.VectorSubcoreMesh(core_axis_name="c", subcore_axis_name="s")
    f = functools.partial(
        pl.kernel,
        mesh=mesh,
        compiler_params=pltpu.CompilerParams(needs_layout_passes=False),
        out_type=jax.ShapeDtypeStruct((_B * 2 * _C * _N * _K,), jnp.float32),
        scratch_types=[
            pltpu.VMEM((_C * _N,), jnp.float32),
            pltpu.VMEM((_PW,), jnp.int32),
            pltpu.VMEM((2 * _C * _PW,), jnp.float32),
        ],
    )(_sc_gather_body)
    out = f(x.reshape(-1), idx.reshape(-1))
    return out.reshape(_B, 2 * _C, _N, _K)


@jax.jit
def kernel(x, class_label):
    del class_label
    idx = _topk_indices(x)
    return _gather_features(x, idx)


# speedup vs baseline: 2.9756x; 1.0443x over previous
import functools

import jax
import jax.numpy as jnp
from jax import lax
from jax.experimental import pallas as pl
from jax.experimental.pallas import tpu as pltpu
from jax.experimental.pallas import tpu_sc as plsc

_K = 40
_B = 4
_C = 3
_N = 4096
_ROWS = 1024

_NC = 2
_NS = 16
_NW = _NC * _NS
_RPW = _N // _NW
_L = 16


def _knn_body(xb_ref, xall_ref, idx_ref):
    xb = xb_ref[0]
    xa = xall_ref[0]
    neg2inner = -2.0 * lax.dot_general(
        xb, xa, (((0,), (0,)), ((), ())), preferred_element_type=jnp.float32
    )
    xx_r = jnp.sum(xb * xb, axis=0)
    xx_c = jnp.sum(xa * xa, axis=0)
    d = (-xx_r[:, None] - neg2inner) - xx_c[None, :]
    iota = lax.broadcasted_iota(jnp.int32, (_ROWS, _N), 1)
    big = jnp.int32(_N)
    neginf = jnp.float32(-jnp.inf)
    for k in range(_K):
        m = jnp.max(d, axis=1)
        cand = jnp.where(d == m[:, None], iota, big)
        amin = jnp.min(cand, axis=1)
        idx_ref[0, :, k : k + 1] = amin[:, None]
        d = jnp.where(cand == amin[:, None], neginf, d)


def _topk_indices(x):
    return pl.pallas_call(
        _knn_body,
        grid=(_B, _N // _ROWS),
        in_specs=[
            pl.BlockSpec((1, _C, _ROWS), lambda b, r: (b, 0, r)),
            pl.BlockSpec((1, _C, _N), lambda b, r: (b, 0, 0)),
        ],
        out_specs=pl.BlockSpec((1, _ROWS, _K), lambda b, r: (b, r, 0)),
        out_shape=jax.ShapeDtypeStruct((_B, _N, _K), jnp.int32),
    )(x, x)


_PW = _RPW * _K


def _sc_gather_body(x_hbm, idx_hbm, out_hbm, table_v, idx_v, out_v):
    wid = lax.axis_index("s") * _NC + lax.axis_index("c")
    n0 = wid * _RPW
    p0 = wid * _PW
    lane = lax.iota(jnp.int32, _L)
    for b in range(_B):
        pltpu.sync_copy(x_hbm.at[pl.ds(b * _C * _N, _C * _N)], table_v)
        pltpu.sync_copy(idx_hbm.at[pl.ds(b * _N * _K + p0, _PW)], idx_v)

        def body(ci, carry):
            base = ci * _L
            pos = base + lane
            r = lax.div(pos, jnp.int32(_K))
            g = r + n0
            nidx = idx_v[pl.ds(base, _L)]
            for c in range(_C):
                off = jnp.int32(c * _N)
                nbr = plsc.load_gather(table_v, [off + nidx])
                ctr = plsc.load_gather(table_v, [off + g])
                out_v[pl.ds(c * _PW + base, _L)] = nbr - ctr
                out_v[pl.ds((c + _C) * _PW + base, _L)] = ctr
            return carry

        lax.fori_loop(0, _PW // _L, body, 0)
        for c in range(2 * _C):
            pltpu.sync_copy(
                out_v.at[pl.ds(c * _PW, _PW)],
                out_hbm.at[pl.ds((b * 2 * _C + c) * _N * _K + p0, _PW)],
            )


def _gather_features(x, idx):
    mesh = plsc.VectorSubcoreMesh(core_axis_name="c", subcore_axis_name="s")
    f = functools.partial(
        pl.kernel,
        mesh=mesh,
        compiler_params=pltpu.CompilerParams(needs_layout_passes=False),
        out_type=jax.ShapeDtypeStruct((_B * 2 * _C * _N * _K,), jnp.float32),
        scratch_types=[
            pltpu.VMEM((_C * _N,), jnp.float32),
            pltpu.VMEM((_PW,), jnp.int32),
            pltpu.VMEM((2 * _C * _PW,), jnp.float32),
        ],
    )(_sc_gather_body)
    out = f(x.reshape(-1), idx.reshape(-1))
    return out.reshape(_B, 2 * _C, _N, _K)


@jax.jit
def kernel(x, class_label):
    del class_label
    idx = _topk_indices(x)
    return _gather_features(x, idx)


# speedup vs baseline: 3.4114x; 1.1465x over previous
import functools

import jax
import jax.numpy as jnp
from jax import lax
from jax.experimental import pallas as pl
from jax.experimental.pallas import tpu as pltpu
from jax.experimental.pallas import tpu_sc as plsc

_K = 40
_B = 4
_C = 3
_N = 4096
_ROWS = 1024

_NC = 2
_NS = 16
_NW = _NC * _NS
_RPW = _N // _NW
_L = 16


def _knn_body(xb_ref, xall_ref, idx_ref):
    xb = xb_ref[0]
    xa = xall_ref[0]
    neg2inner = -2.0 * lax.dot_general(
        xb, xa, (((0,), (0,)), ((), ())), preferred_element_type=jnp.float32
    )
    xx_r = jnp.sum(xb * xb, axis=0)
    xx_c = jnp.sum(xa * xa, axis=0)
    d = (-xx_r[:, None] - neg2inner) - xx_c[None, :]
    big = jnp.int32(1 << 30)
    neginf = jnp.float32(-jnp.inf)
    h = _N // 2
    fh = d[:, :h]
    sh = d[:, h:]
    iota = lax.broadcasted_iota(jnp.int32, (_ROWS, h), 1)
    fge = fh >= sh
    p = jnp.where(fge, fh, sh)
    g = jnp.where(fge, iota, iota + h)
    p2 = jnp.where(fge, sh, fh)
    g2 = jnp.where(fge, iota + h, iota)
    for k in range(_K):
        m = jnp.max(p, axis=1)
        amin = jnp.min(jnp.where(p == m[:, None], g, big), axis=1)
        idx_ref[0, :, k : k + 1] = amin[:, None]
        slot = g == amin[:, None]
        p = jnp.where(slot, p2, p)
        g = jnp.where(slot, g2, g)
        p2 = jnp.where(slot, neginf, p2)


def _topk_indices(x):
    return pl.pallas_call(
        _knn_body,
        grid=(_B, _N // _ROWS),
        in_specs=[
            pl.BlockSpec((1, _C, _ROWS), lambda b, r: (b, 0, r)),
            pl.BlockSpec((1, _C, _N), lambda b, r: (b, 0, 0)),
        ],
        out_specs=pl.BlockSpec((1, _ROWS, _K), lambda b, r: (b, r, 0)),
        out_shape=jax.ShapeDtypeStruct((_B, _N, _K), jnp.int32),
    )(x, x)


_PW = _RPW * _K


def _sc_gather_body(x_hbm, idx_hbm, out_hbm, table_v, idx_v, out_v):
    wid = lax.axis_index("s") * _NC + lax.axis_index("c")
    n0 = wid * _RPW
    p0 = wid * _PW
    lane = lax.iota(jnp.int32, _L)
    for b in range(_B):
        pltpu.sync_copy(x_hbm.at[pl.ds(b * _C * _N, _C * _N)], table_v)
        pltpu.sync_copy(idx_hbm.at[pl.ds(b * _N * _K + p0, _PW)], idx_v)

        def body(ci, carry):
            base = ci * _L
            pos = base + lane
            r = lax.div(pos, jnp.int32(_K))
            g = r + n0
            nidx = idx_v[pl.ds(base, _L)]
            for c in range(_C):
                off = jnp.int32(c * _N)
                nbr = plsc.load_gather(table_v, [off + nidx])
                ctr = plsc.load_gather(table_v, [off + g])
                out_v[pl.ds(c * _PW + base, _L)] = nbr - ctr
                out_v[pl.ds((c + _C) * _PW + base, _L)] = ctr
            return carry

        lax.fori_loop(0, _PW // _L, body, 0)
        for c in range(2 * _C):
            pltpu.sync_copy(
                out_v.at[pl.ds(c * _PW, _PW)],
                out_hbm.at[pl.ds((b * 2 * _C + c) * _N * _K + p0, _PW)],
            )


def _gather_features(x, idx):
    mesh = plsc.VectorSubcoreMesh(core_axis_name="c", subcore_axis_name="s")
    f = functools.partial(
        pl.kernel,
        mesh=mesh,
        compiler_params=pltpu.CompilerParams(needs_layout_passes=False),
        out_type=jax.ShapeDtypeStruct((_B * 2 * _C * _N * _K,), jnp.float32),
        scratch_types=[
            pltpu.VMEM((_C * _N,), jnp.float32),
            pltpu.VMEM((_PW,), jnp.int32),
            pltpu.VMEM((2 * _C * _PW,), jnp.float32),
        ],
    )(_sc_gather_body)
    out = f(x.reshape(-1), idx.reshape(-1))
    return out.reshape(_B, 2 * _C, _N, _K)


@jax.jit
def kernel(x, class_label):
    del class_label
    idx = _topk_indices(x)
    return _gather_features(x, idx)
